# baseline (speedup 1.0000x reference)
import functools

import jax
import jax.numpy as jnp
from jax import lax
from jax.experimental import pallas as pl
from jax.experimental.pallas import tpu as pltpu
from jax.experimental.pallas import tpu_sc as plsc

B = 4
L = 16384
N = B * L
C_IN = 6
NUM_CLASSES = 13
BETA = 0.999

SC_CORES = 2
SC_SUBCORES = 16
CHUNK = 128

R = 8192
NBLK = N // R


def _dotb(a, b):
    return jnp.dot(a.astype(jnp.bfloat16), b,
                   preferred_element_type=jnp.float32)


def _shift_add_cumsum(a, axis, length):
    sh = 1
    while sh < length:
        zeros_idx = [slice(None)] * a.ndim
        keep_idx = [slice(None)] * a.ndim
        zeros_idx[axis] = slice(0, sh)
        keep_idx[axis] = slice(0, length - sh)
        shifted = jnp.concatenate(
            [jnp.zeros_like(a[tuple(zeros_idx)]), a[tuple(keep_idx)]], axis=axis)
        a = a + shifted
        sh *= 2
    return a


def _dest_counts_body(y_ref, dest_ref, counts_ref):
    y = y_ref[...]
    cls = lax.broadcasted_iota(jnp.int32, (1, NUM_CLASSES, 1, 1), 1)
    oh0 = (y[:, None, :, :] == cls).astype(jnp.float32)
    r_i = lax.broadcasted_iota(jnp.int32, (128, 128), 0)
    c_i = lax.broadcasted_iota(jnp.int32, (128, 128), 1)
    tri_incl = (r_i <= c_i).astype(jnp.bfloat16)
    tri_excl = (r_i < c_i).astype(jnp.bfloat16)
    ohb = oh0.astype(jnp.bfloat16).reshape(B * NUM_CLASSES * 128, 128)
    intra = jnp.dot(ohb, tri_incl, preferred_element_type=jnp.float32)
    intra = intra.reshape(B, NUM_CLASSES, 128, 128)
    rowtot = intra[:, :, :, 127:128]
    rt2 = rowtot.astype(jnp.bfloat16).reshape(B * NUM_CLASSES, 128)
    carries = jnp.dot(rt2, tri_excl, preferred_element_type=jnp.float32)
    carries = carries.reshape(B, NUM_CLASSES, 128)
    prefix = intra + carries[:, :, :, None]
    counts = jnp.sum(rowtot[:, :, :, 0], axis=2)
    counts_ref[...] = counts
    offs = _shift_add_cumsum(counts, axis=1, length=NUM_CLASSES) - counts
    base = lax.broadcasted_iota(jnp.int32, (B, 1), 0).astype(jnp.float32) * float(L)
    offs = offs + base
    val = offs[:, :, None, None] + prefix - 1.0
    dest = jnp.sum(oh0 * val, axis=1).astype(jnp.int32)
    k_i = lax.broadcasted_iota(jnp.int32, (1, C_IN, 1, 1), 1)
    dest_ref[...] = dest[:, None, :, :] * C_IN + k_i


def _dest_counts(y3):
    return pl.pallas_call(
        _dest_counts_body,
        out_shape=[
            jax.ShapeDtypeStruct((B, C_IN, 128, 128), jnp.int32),
            jax.ShapeDtypeStruct((B, NUM_CLASSES), jnp.float32),
        ],
    )(y3)


NE = N * C_IN
E_PER_SUB = NE // SC_SUBCORES
E_CHUNKS = E_PER_SUB // CHUNK


def _scatter_feat_body(x_hbm, idx_hbm, out_hbm, idx_v, val_v, shared, sem):
    cid = lax.axis_index("c")
    sid = lax.axis_index("s")

    @pl.when(cid == 0)
    def _():
        pltpu.sync_copy(idx_hbm.at[sid], idx_v)
        pltpu.sync_copy(x_hbm.at[sid], val_v)

        @pl.loop(0, E_CHUNKS)
        def _(j):
            pltpu.async_copy(val_v.at[j], shared.at[idx_v.at[j]], sem)

        @pl.loop(0, E_CHUNKS)
        def _(j):
            pltpu.make_async_copy(x_hbm.at[sid].at[j], val_v.at[j], sem).wait()

    plsc.subcore_barrier()

    @pl.when(cid == 0)
    def _():
        pltpu.sync_copy(shared.at[pl.ds(sid * E_PER_SUB, E_PER_SUB)],
                        out_hbm.at[pl.ds(sid * E_PER_SUB, E_PER_SUB)])


def _scatter_feat(x_sc, idx_sc):
    mesh = plsc.VectorSubcoreMesh(core_axis_name="c", subcore_axis_name="s")
    kern = functools.partial(
        pl.kernel,
        mesh=mesh,
        out_type=jax.ShapeDtypeStruct((NE,), jnp.float32),
        scratch_types=[
            pltpu.VMEM((E_CHUNKS, CHUNK), jnp.int32),
            pltpu.VMEM((E_CHUNKS, CHUNK), jnp.float32),
            pltpu.VMEM_SHARED((NE,), jnp.float32),
            pltpu.SemaphoreType.DMA,
        ],
    )(_scatter_feat_body)
    return kern(x_sc, idx_sc)


def _sweep1_body(feat_ref, wpe_ref, bpe_ref, w1_ref, b1_ref, w2_ref, b2_ref,
                 w3_ref, b3_ref, wp1_ref, bp1_ref, z1_ref, st_ref):
    i = pl.program_id(0)
    f = feat_ref[...]
    h = jax.nn.relu(_dotb(f, wpe_ref[...]) + bpe_ref[...])
    h = jax.nn.relu(_dotb(h, w1_ref[...]) + b1_ref[...])
    h = jax.nn.relu(_dotb(h, w2_ref[...]) + b2_ref[...])
    h = jax.nn.relu(_dotb(h, w3_ref[...]) + b3_ref[...])
    z1 = _dotb(h, wp1_ref[...]) + bp1_ref[...]
    z1_ref[...] = z1.astype(jnp.bfloat16)
    s = jnp.sum(z1, axis=0, keepdims=True)
    s2 = jnp.sum(z1 * z1, axis=0, keepdims=True)

    @pl.when(i == 0)
    def _():
        st_ref[...] = jnp.zeros_like(st_ref)

    st_ref[...] += jnp.concatenate([s, s2], axis=0)


def _sweep1(feat, wpe, bpe, w1, b1, w2, b2, w3, b3, wp1, bp1):
    def _c(shape):
        return pl.BlockSpec(shape, lambda i: (0, 0))

    return pl.pallas_call(
        _sweep1_body,
        grid=(NBLK,),
        in_specs=[
            pl.BlockSpec((R, C_IN), lambda i: (i, 0)),
            _c((C_IN, 48)), _c((1, 48)),
            _c((48, 96)), _c((1, 96)),
            _c((96, 192)), _c((1, 192)),
            _c((192, 384)), _c((1, 384)),
            _c((384, 192)), _c((1, 192)),
        ],
        out_specs=[
            pl.BlockSpec((R, 192), lambda i: (i, 0)),
            pl.BlockSpec((2, 192), lambda i: (0, 0)),
        ],
        out_shape=[
            jax.ShapeDtypeStruct((N, 192), jnp.bfloat16),
            jax.ShapeDtypeStruct((2, 192), jnp.float32),
        ],
    )(feat, wpe, bpe, w1, b1, w2, b2, w3, b3, wp1, bp1)


def _sweep23_body(z1_ref, st_ref, wp2_ref, bp2_ref, counts_ref, prior_ref,
                  cp_ref, pe_ref, z2a_ref, st2_ref, csum_ref, cnt_ref):
    p = pl.program_id(0)
    i = pl.program_id(1)

    @pl.when(p == 0)
    def _():
        st = st_ref[...]
        m = st[0:1, :] / float(N)
        v = st[1:2, :] / float(N) - m * m
        inv = lax.rsqrt(v + 1e-5)
        z1n = jax.nn.relu((z1_ref[...].astype(jnp.float32) - m) * inv)
        z2 = _dotb(z1n, wp2_ref[...]) + bp2_ref[...]
        z2a_ref[pl.ds(i * R, R), :] = z2.astype(jnp.bfloat16)
        s = jnp.sum(z2, axis=0, keepdims=True)
        s2 = jnp.sum(z2 * z2, axis=0, keepdims=True)

        @pl.when(i == 0)
        def _():
            st2_ref[...] = jnp.zeros_like(st2_ref)

        st2_ref[...] += jnp.concatenate([s, s2], axis=0)

    @pl.when(p == 1)
    def _():
        st = st2_ref[...]
        m = st[0:1, :] / float(N)
        v = st[1:2, :] / float(N) - m * m
        inv = lax.rsqrt(v + 1e-5)
        z2 = z2a_ref[pl.ds(i * R, R), :].astype(jnp.float32)
        z2n = jax.nn.relu((z2 - m) * inv)
        norm = jnp.sqrt(jnp.sum(z2n * z2n, axis=1, keepdims=True))
        h = z2n / (norm + 1e-12)

        start = i * R
        b = start // L
        j_local = (lax.broadcasted_iota(jnp.int32, (R, 1), 0)
                   .astype(jnp.float32) + (start % L).astype(jnp.float32))
        counts_b = counts_ref[pl.ds(b, 1), :]
        cum_incl = _shift_add_cumsum(counts_b, axis=1, length=NUM_CLASSES)
        lab = jnp.sum((j_local >= cum_incl).astype(jnp.float32), axis=1,
                      keepdims=True)
        cp_ref[...] = jnp.concatenate([h, lab], axis=1)

        cls = (lax.broadcasted_iota(jnp.int32, (1, NUM_CLASSES), 1)
               .astype(jnp.float32))
        onehot = (lab == cls).astype(jnp.float32)

        @pl.when(i == 0)
        def _():
            csum_ref[...] = jnp.zeros_like(csum_ref)
            cnt_ref[...] = jnp.zeros_like(cnt_ref)

        dn = (((0,), (0,)), ((), ()))
        csum_ref[...] += lax.dot_general(onehot, h, dn,
                                         preferred_element_type=jnp.float32)
        cnt_ref[...] += lax.dot_general(onehot, jnp.ones((R, 1), jnp.float32),
                                        dn, preferred_element_type=jnp.float32)

        @pl.when(i == NBLK - 1)
        def _():
            cnt = cnt_ref[...]
            means = csum_ref[...] / jnp.maximum(cnt, 1.0)
            prior = prior_ref[...]
            cur = jnp.where(cnt > 0, means, prior)
            pe = BETA * prior + (1.0 - BETA) * cur
            pe_norm = jnp.sqrt(jnp.sum(pe * pe, axis=1, keepdims=True))
            pe_ref[...] = pe / pe_norm


def _sweep23(z1, st1, wp2, bp2, counts, prior):
    def _c(shape):
        return pl.BlockSpec(shape, lambda p, i: (0, 0))

    return pl.pallas_call(
        _sweep23_body,
        grid=(2, NBLK),
        in_specs=[
            pl.BlockSpec((R, 192), lambda p, i: (jnp.where(p == 0, i, 0), 0)),
            _c((2, 192)), _c((192, 48)), _c((1, 48)),
            _c((B, NUM_CLASSES)), _c((NUM_CLASSES, 48)),
        ],
        out_specs=[
            pl.BlockSpec((R, 49), lambda p, i: (jnp.where(p == 1, i, 0), 0)),
            pl.BlockSpec((NUM_CLASSES, 48), lambda p, i: (0, 0)),
        ],
        out_shape=[
            jax.ShapeDtypeStruct((N, 49), jnp.float32),
            jax.ShapeDtypeStruct((NUM_CLASSES, 48), jnp.float32),
        ],
        scratch_shapes=[
            pltpu.VMEM((N, 48), jnp.bfloat16),
            pltpu.VMEM((2, 48), jnp.float32),
            pltpu.VMEM((NUM_CLASSES, 48), jnp.float32),
            pltpu.VMEM((NUM_CLASSES, 1), jnp.float32),
        ],
    )(z1, st1, wp2, bp2, counts, prior)


def kernel(pos, x, y, W_pe, b_pe, W1, b1, W2, b2, W3, b3, Wp1, bp1, Wp2, bp2,
           prior_ema):
    y3 = y.astype(jnp.int32).reshape(B, 128, 128)
    dest3, counts = _dest_counts(y3)

    x_sc = x.reshape(SC_SUBCORES, E_CHUNKS, CHUNK)
    idx_sc = dest3.reshape(SC_SUBCORES, E_CHUNKS, CHUNK)
    feat_s = _scatter_feat(x_sc, idx_sc).reshape(N, C_IN)

    bf = jnp.bfloat16
    z1, st1 = _sweep1(feat_s.astype(bf), W_pe.astype(bf),
                      b_pe.reshape(1, -1).astype(bf),
                      W1.astype(bf), b1.reshape(1, -1).astype(bf),
                      W2.astype(bf), b2.reshape(1, -1).astype(bf),
                      W3.astype(bf), b3.reshape(1, -1).astype(bf),
                      Wp1.astype(bf), bp1.reshape(1, -1).astype(bf))
    cp, pe = _sweep23(z1, st1, Wp2.astype(bf), bp2.reshape(1, -1).astype(bf),
                      counts, prior_ema)
    return cp, pe


# speedup vs baseline: 1.0030x; 1.0030x over previous
import functools

import jax
import jax.numpy as jnp
from jax import lax
from jax.experimental import pallas as pl
from jax.experimental.pallas import tpu as pltpu
from jax.experimental.pallas import tpu_sc as plsc

B = 4
L = 16384
N = B * L
C_IN = 6
NUM_CLASSES = 13
BETA = 0.999

SC_CORES = 2
SC_SUBCORES = 16
CHUNK = 128

R = 8192
NBLK = N // R


def _dotb(a, b):
    return jnp.dot(a.astype(jnp.bfloat16), b,
                   preferred_element_type=jnp.float32)


def _shift_add_cumsum(a, axis, length):
    sh = 1
    while sh < length:
        zeros_idx = [slice(None)] * a.ndim
        keep_idx = [slice(None)] * a.ndim
        zeros_idx[axis] = slice(0, sh)
        keep_idx[axis] = slice(0, length - sh)
        shifted = jnp.concatenate(
            [jnp.zeros_like(a[tuple(zeros_idx)]), a[tuple(keep_idx)]], axis=axis)
        a = a + shifted
        sh *= 2
    return a


def _dest_counts_body(y_ref, dest_ref, counts_ref):
    y = y_ref[...]
    cls = lax.broadcasted_iota(jnp.int32, (1, NUM_CLASSES, 1, 1), 1)
    oh0 = (y[:, None, :, :] == cls).astype(jnp.float32)
    r_i = lax.broadcasted_iota(jnp.int32, (128, 128), 0)
    c_i = lax.broadcasted_iota(jnp.int32, (128, 128), 1)
    tri_incl = (r_i <= c_i).astype(jnp.bfloat16)
    tri_excl = (r_i < c_i).astype(jnp.bfloat16)
    ohb = oh0.astype(jnp.bfloat16).reshape(B * NUM_CLASSES * 128, 128)
    intra = jnp.dot(ohb, tri_incl, preferred_element_type=jnp.float32)
    intra = intra.reshape(B, NUM_CLASSES, 128, 128)
    rowtot = intra[:, :, :, 127:128]
    rt2 = rowtot.astype(jnp.bfloat16).reshape(B * NUM_CLASSES, 128)
    carries = jnp.dot(rt2, tri_excl, preferred_element_type=jnp.float32)
    carries = carries.reshape(B, NUM_CLASSES, 128)
    prefix = intra + carries[:, :, :, None]
    counts = jnp.sum(rowtot[:, :, :, 0], axis=2)
    counts_ref[...] = counts
    offs = _shift_add_cumsum(counts, axis=1, length=NUM_CLASSES) - counts
    base = lax.broadcasted_iota(jnp.int32, (B, 1), 0).astype(jnp.float32) * float(L)
    offs = offs + base
    val = offs[:, :, None, None] + prefix - 1.0
    dest = jnp.sum(oh0 * val, axis=1).astype(jnp.int32)
    k_i = lax.broadcasted_iota(jnp.int32, (1, C_IN, 1, 1), 1)
    dest_ref[...] = dest[:, None, :, :] * C_IN + k_i


def _dest_counts(y3):
    return pl.pallas_call(
        _dest_counts_body,
        out_shape=[
            jax.ShapeDtypeStruct((B, C_IN, 128, 128), jnp.int32),
            jax.ShapeDtypeStruct((B, NUM_CLASSES), jnp.float32),
        ],
    )(y3)


NE = N * C_IN
E_PER_SUB = NE // SC_SUBCORES
E_CHUNKS = E_PER_SUB // CHUNK


def _scatter_feat_body(x_hbm, idx_hbm, out_hbm, idx_v, val_v, shared, sem):
    cid = lax.axis_index("c")
    sid = lax.axis_index("s")

    @pl.when(cid == 0)
    def _():
        pltpu.async_copy(idx_hbm.at[sid], idx_v, sem)
        pltpu.async_copy(x_hbm.at[sid], val_v, sem)
        pltpu.make_async_copy(idx_hbm.at[sid], idx_v, sem).wait()
        pltpu.make_async_copy(x_hbm.at[sid], val_v, sem).wait()

        @pl.loop(0, E_CHUNKS)
        def _(j):
            pltpu.async_copy(val_v.at[j], shared.at[idx_v.at[j]], sem)

        @pl.loop(0, E_CHUNKS)
        def _(j):
            pltpu.make_async_copy(x_hbm.at[sid].at[j], val_v.at[j], sem).wait()

    plsc.subcore_barrier()

    @pl.when(cid == 0)
    def _():
        pltpu.sync_copy(shared.at[pl.ds(sid * E_PER_SUB, E_PER_SUB)],
                        out_hbm.at[pl.ds(sid * E_PER_SUB, E_PER_SUB)])


def _scatter_feat(x_sc, idx_sc):
    mesh = plsc.VectorSubcoreMesh(core_axis_name="c", subcore_axis_name="s")
    kern = functools.partial(
        pl.kernel,
        mesh=mesh,
        out_type=jax.ShapeDtypeStruct((NE,), jnp.float32),
        scratch_types=[
            pltpu.VMEM((E_CHUNKS, CHUNK), jnp.int32),
            pltpu.VMEM((E_CHUNKS, CHUNK), jnp.float32),
            pltpu.VMEM_SHARED((NE,), jnp.float32),
            pltpu.SemaphoreType.DMA,
        ],
    )(_scatter_feat_body)
    return kern(x_sc, idx_sc)


def _sweep1_body(feat_ref, wpe_ref, bpe_ref, w1_ref, b1_ref, w2_ref, b2_ref,
                 w3_ref, b3_ref, wp1_ref, bp1_ref, z1_ref, st_ref):
    i = pl.program_id(0)
    f = feat_ref[...]
    h = jax.nn.relu(_dotb(f, wpe_ref[...]) + bpe_ref[...])
    h = jax.nn.relu(_dotb(h, w1_ref[...]) + b1_ref[...])
    h = jax.nn.relu(_dotb(h, w2_ref[...]) + b2_ref[...])
    h = jax.nn.relu(_dotb(h, w3_ref[...]) + b3_ref[...])
    z1 = _dotb(h, wp1_ref[...]) + bp1_ref[...]
    z1_ref[...] = z1.astype(jnp.bfloat16)
    s = jnp.sum(z1, axis=0, keepdims=True)
    s2 = jnp.sum(z1 * z1, axis=0, keepdims=True)

    @pl.when(i == 0)
    def _():
        st_ref[...] = jnp.zeros_like(st_ref)

    st_ref[...] += jnp.concatenate([s, s2], axis=0)


def _sweep1(feat, wpe, bpe, w1, b1, w2, b2, w3, b3, wp1, bp1):
    def _c(shape):
        return pl.BlockSpec(shape, lambda i: (0, 0))

    return pl.pallas_call(
        _sweep1_body,
        grid=(NBLK,),
        in_specs=[
            pl.BlockSpec((R, C_IN), lambda i: (i, 0)),
            _c((C_IN, 48)), _c((1, 48)),
            _c((48, 96)), _c((1, 96)),
            _c((96, 192)), _c((1, 192)),
            _c((192, 384)), _c((1, 384)),
            _c((384, 192)), _c((1, 192)),
        ],
        out_specs=[
            pl.BlockSpec((R, 192), lambda i: (i, 0)),
            pl.BlockSpec((2, 192), lambda i: (0, 0)),
        ],
        out_shape=[
            jax.ShapeDtypeStruct((N, 192), jnp.bfloat16),
            jax.ShapeDtypeStruct((2, 192), jnp.float32),
        ],
    )(feat, wpe, bpe, w1, b1, w2, b2, w3, b3, wp1, bp1)


def _sweep23_body(z1_ref, st_ref, wp2_ref, bp2_ref, counts_ref, prior_ref,
                  cp_ref, pe_ref, z2a_ref, st2_ref, csum_ref, cnt_ref):
    p = pl.program_id(0)
    i = pl.program_id(1)

    @pl.when(p == 0)
    def _():
        st = st_ref[...]
        m = st[0:1, :] / float(N)
        v = st[1:2, :] / float(N) - m * m
        inv = lax.rsqrt(v + 1e-5)
        z1n = jax.nn.relu((z1_ref[...].astype(jnp.float32) - m) * inv)
        z2 = _dotb(z1n, wp2_ref[...]) + bp2_ref[...]
        z2a_ref[pl.ds(i * R, R), :] = z2.astype(jnp.bfloat16)
        s = jnp.sum(z2, axis=0, keepdims=True)
        s2 = jnp.sum(z2 * z2, axis=0, keepdims=True)

        @pl.when(i == 0)
        def _():
            st2_ref[...] = jnp.zeros_like(st2_ref)

        st2_ref[...] += jnp.concatenate([s, s2], axis=0)

    @pl.when(p == 1)
    def _():
        st = st2_ref[...]
        m = st[0:1, :] / float(N)
        v = st[1:2, :] / float(N) - m * m
        inv = lax.rsqrt(v + 1e-5)
        z2 = z2a_ref[pl.ds(i * R, R), :].astype(jnp.float32)
        z2n = jax.nn.relu((z2 - m) * inv)
        norm = jnp.sqrt(jnp.sum(z2n * z2n, axis=1, keepdims=True))
        h = z2n / (norm + 1e-12)

        start = i * R
        b = start // L
        j_local = (lax.broadcasted_iota(jnp.int32, (R, 1), 0)
                   .astype(jnp.float32) + (start % L).astype(jnp.float32))
        counts_b = counts_ref[pl.ds(b, 1), :]
        cum_incl = _shift_add_cumsum(counts_b, axis=1, length=NUM_CLASSES)
        lab = jnp.sum((j_local >= cum_incl).astype(jnp.float32), axis=1,
                      keepdims=True)
        cp_ref[...] = jnp.concatenate([h, lab], axis=1)

        cls = (lax.broadcasted_iota(jnp.int32, (1, NUM_CLASSES), 1)
               .astype(jnp.float32))
        onehot = (lab == cls).astype(jnp.float32)

        @pl.when(i == 0)
        def _():
            csum_ref[...] = jnp.zeros_like(csum_ref)
            cnt_ref[...] = jnp.zeros_like(cnt_ref)

        dn = (((0,), (0,)), ((), ()))
        csum_ref[...] += lax.dot_general(onehot, h, dn,
                                         preferred_element_type=jnp.float32)
        cnt_ref[...] += lax.dot_general(onehot, jnp.ones((R, 1), jnp.float32),
                                        dn, preferred_element_type=jnp.float32)

        @pl.when(i == NBLK - 1)
        def _():
            cnt = cnt_ref[...]
            means = csum_ref[...] / jnp.maximum(cnt, 1.0)
            prior = prior_ref[...]
            cur = jnp.where(cnt > 0, means, prior)
            pe = BETA * prior + (1.0 - BETA) * cur
            pe_norm = jnp.sqrt(jnp.sum(pe * pe, axis=1, keepdims=True))
            pe_ref[...] = pe / pe_norm


def _sweep23(z1, st1, wp2, bp2, counts, prior):
    def _c(shape):
        return pl.BlockSpec(shape, lambda p, i: (0, 0))

    return pl.pallas_call(
        _sweep23_body,
        grid=(2, NBLK),
        in_specs=[
            pl.BlockSpec((R, 192), lambda p, i: (jnp.where(p == 0, i, 0), 0)),
            _c((2, 192)), _c((192, 48)), _c((1, 48)),
            _c((B, NUM_CLASSES)), _c((NUM_CLASSES, 48)),
        ],
        out_specs=[
            pl.BlockSpec((R, 49), lambda p, i: (jnp.where(p == 1, i, 0), 0)),
            pl.BlockSpec((NUM_CLASSES, 48), lambda p, i: (0, 0)),
        ],
        out_shape=[
            jax.ShapeDtypeStruct((N, 49), jnp.float32),
            jax.ShapeDtypeStruct((NUM_CLASSES, 48), jnp.float32),
        ],
        scratch_shapes=[
            pltpu.VMEM((N, 48), jnp.bfloat16),
            pltpu.VMEM((2, 48), jnp.float32),
            pltpu.VMEM((NUM_CLASSES, 48), jnp.float32),
            pltpu.VMEM((NUM_CLASSES, 1), jnp.float32),
        ],
    )(z1, st1, wp2, bp2, counts, prior)


def kernel(pos, x, y, W_pe, b_pe, W1, b1, W2, b2, W3, b3, Wp1, bp1, Wp2, bp2,
           prior_ema):
    y3 = y.astype(jnp.int32).reshape(B, 128, 128)
    dest3, counts = _dest_counts(y3)

    x_sc = x.reshape(SC_SUBCORES, E_CHUNKS, CHUNK)
    idx_sc = dest3.reshape(SC_SUBCORES, E_CHUNKS, CHUNK)
    feat_s = _scatter_feat(x_sc, idx_sc).reshape(N, C_IN)

    bf = jnp.bfloat16
    z1, st1 = _sweep1(feat_s.astype(bf), W_pe.astype(bf),
                      b_pe.reshape(1, -1).astype(bf),
                      W1.astype(bf), b1.reshape(1, -1).astype(bf),
                      W2.astype(bf), b2.reshape(1, -1).astype(bf),
                      W3.astype(bf), b3.reshape(1, -1).astype(bf),
                      Wp1.astype(bf), bp1.reshape(1, -1).astype(bf))
    cp, pe = _sweep23(z1, st1, Wp2.astype(bf), bp2.reshape(1, -1).astype(bf),
                      counts, prior_ema)
    return cp, pe
